# baseline (speedup 1.0000x reference)
import functools
import jax
import jax.numpy as jnp
from jax import lax
from jax.experimental import pallas as pl
from jax.experimental.pallas import tpu as pltpu
from jax.experimental.pallas import tpu_sc as plsc

B = 4096
L = 768
E = 8
P = 48
PD = 16
F = 32

NC = 2
NS = 16
NW = NC * NS
TW = B // NW
NG = TW // 16

GW_O = 0
GB_O = 128
EB_O = 144
EW_O = 144 + 1024
WTS = EW_O + E * 2 * F * PD

A0_O = 0
A1_O = 128
S0_O = 256
S1_O = 272
ABUF = 288


def _sc_body(x_hbm, noise_hbm, wts_hbm, out_hbm, disp_hbm, parts_hbm,
             xs, ns, ws, ab, acc, ob, db, sem):
    c = lax.axis_index("c")
    s = lax.axis_index("s")
    wid = s * NC + c
    base = wid * TW

    cp = pltpu.async_copy(x_hbm.at[pl.ds(base * L, TW * L)], xs, sem)
    pltpu.sync_copy(wts_hbm, ws)
    pltpu.sync_copy(noise_hbm.at[pl.ds(base, TW)], ns)

    iota = lax.iota(jnp.int32, 16)
    zero = jnp.zeros((16,), jnp.float32)

    def prep_e(e, _):
        a0 = zero
        a1 = zero
        for ch in range(F):
            a0 = a0 + ws[pl.ds(EW_O + e * 1024 + ch * 16, 16)]
            a1 = a1 + ws[pl.ds(EW_O + e * 1024 + (F + ch) * 16, 16)]
        ab[pl.ds(A0_O + e * 16, 16)] = a0
        ab[pl.ds(A1_O + e * 16, 16)] = a1
        return 0

    lax.fori_loop(0, E, prep_e, 0)

    row64 = EB_O + iota * 64

    def prep_s(ch, carry):
        s0, s1 = carry
        s0 = s0 + plsc.load_gather(ws, [row64 + ch])
        s1 = s1 + plsc.load_gather(ws, [row64 + (F + ch)])
        return (s0, s1)

    s0, s1 = lax.fori_loop(0, F, prep_s, (zero, zero))
    ab[pl.ds(S0_O, 16)] = s0 * float(P)
    ab[pl.ds(S1_O, 16)] = s1 * float(P)

    for i in range(16):
        acc[pl.ds(i * 16, 16)] = zero

    cp.wait()

    def group(g, _):
        gi = g * 16
        row_off = (gi + iota) * L
        vt = []
        for k in range(PD):
            a = plsc.load_gather(xs, [row_off + k])
            for p in range(1, P):
                a = a + plsc.load_gather(xs, [row_off + (p * 16 + k)])
            vt.append(a)

        nz = ns[pl.ds(gi, 16)]
        hs = []
        for e in range(E):
            he = plsc.load_gather(ws, [jnp.full((16,), GB_O + e, jnp.int32)])
            he = he * float(P) + nz
            for k in range(PD):
                w = plsc.load_gather(ws, [jnp.full((16,), GW_O + e * 16 + k, jnp.int32)])
                he = he + w * vt[k]
            hs.append(he)

        m = hs[0]
        ei = jnp.zeros((16,), jnp.int32)
        for e in range(1, E):
            gt = hs[e] > m
            m = jnp.where(gt, hs[e], m)
            ei = jnp.where(gt, e, ei)

        for e in range(E):
            de = jnp.where(ei == e, 1.0, 0.0).astype(jnp.float32)
            plsc.store_scatter(db, [iota * 8 + (g * 128 + e)], de)
            acc[pl.ds(e * 16, 16)] = acc[pl.ds(e * 16, 16)] + hs[e]
            acc[pl.ds(128 + e * 16, 16)] = acc[pl.ds(128 + e * 16, 16)] + de

        t0 = plsc.load_gather(ab, [S0_O + ei])
        t1 = plsc.load_gather(ab, [S1_O + ei])
        ei16 = ei * 16
        for k in range(PD):
            t0 = t0 + vt[k] * plsc.load_gather(ab, [A0_O + ei16 + k])
            t1 = t1 + vt[k] * plsc.load_gather(ab, [A1_O + ei16 + k])
        plsc.store_scatter(ob, [iota * 2 + (g * 32)], m * t0)
        plsc.store_scatter(ob, [iota * 2 + (g * 32 + 1)], m * t1)
        return 0

    lax.fori_loop(0, NG, group, 0)

    pltpu.sync_copy(ob, out_hbm.at[pl.ds(wid * 256, 256)])
    pltpu.sync_copy(db, disp_hbm.at[pl.ds(wid * 1024, 1024)])
    pltpu.sync_copy(acc, parts_hbm.at[pl.ds(wid * 256, 256)])


_sc_call = pl.kernel(
    _sc_body,
    out_type=[
        jax.ShapeDtypeStruct((B * 2,), jnp.float32),
        jax.ShapeDtypeStruct((B * E,), jnp.float32),
        jax.ShapeDtypeStruct((NW * 256,), jnp.float32),
    ],
    mesh=plsc.VectorSubcoreMesh(core_axis_name="c", subcore_axis_name="s"),
    compiler_params=pltpu.CompilerParams(needs_layout_passes=False),
    scratch_types=[
        pltpu.VMEM((TW * L,), jnp.float32),
        pltpu.VMEM((TW,), jnp.float32),
        pltpu.VMEM((WTS,), jnp.float32),
        pltpu.VMEM((ABUF,), jnp.float32),
        pltpu.VMEM((256,), jnp.float32),
        pltpu.VMEM((256,), jnp.float32),
        pltpu.VMEM((1024,), jnp.float32),
        pltpu.SemaphoreType.DMA,
    ],
)


def _fin_body(parts_ref, loss_ref):
    a = parts_ref[...]
    ssum = jnp.sum(a, axis=0)[None, :]
    r = lax.broadcasted_iota(jnp.int32, (256, 16), 0)
    cc = lax.broadcasted_iota(jnp.int32, (256, 16), 1)
    sel = (((r // 16) % 8 == cc % 8) & ((r < 128) == (cc < 8))).astype(jnp.float32)
    s2 = jnp.dot(ssum, sel, preferred_element_type=jnp.float32)
    i2 = lax.broadcasted_iota(jnp.int32, (16, 16), 0)
    j2 = lax.broadcasted_iota(jnp.int32, (16, 16), 1)
    sh = (i2 == j2 + 8).astype(jnp.float32)
    s3 = jnp.dot(s2, sh, preferred_element_type=jnp.float32)
    loss_ref[...] = (float(E) / float(B * B)) * jnp.sum(s2 * s3).reshape(1, 1)


def _finalize(parts2):
    return pl.pallas_call(
        _fin_body,
        out_shape=jax.ShapeDtypeStruct((1, 1), jnp.float32),
    )(parts2)


@jax.jit
def _run(xf, noise, wts):
    out_f, disp_f, parts = _sc_call(xf, noise, wts)
    loss = _finalize(parts.reshape(NW, 256))
    return out_f.reshape(B, 2), disp_f.reshape(B, E), loss[0, 0]


def kernel(x, gw, gb, ew, eb):
    b = x.shape[0]
    xf = x.reshape(b * L)
    noise = jax.random.uniform(jax.random.key(42), (b, 1), dtype=jnp.float32).reshape(b)
    gb16 = jnp.pad(gb, (0, 8))
    eb16 = jnp.pad(eb.reshape(E * 2 * F), (0, 512))
    wts = jnp.concatenate([
        gw[:, 0, :].reshape(E * PD),
        gb16,
        eb16,
        ew[:, :, 0, :].reshape(E * 2 * F * PD),
    ])
    return _run(xf, noise, wts)


# speedup vs baseline: 1.5180x; 1.5180x over previous
import functools
import jax
import jax.numpy as jnp
from jax import lax
from jax.experimental import pallas as pl
from jax.experimental.pallas import tpu as pltpu
from jax.experimental.pallas import tpu_sc as plsc

B = 4096
L = 768
E = 8
P = 48
PD = 16
F = 32

NC = 2
NS = 16
NW = NC * NS
TW = B // NW
NG = TW // 16

GW_O = 0
GB_O = 128
EB_O = 144
EW_O = 144 + 1024
WTS = EW_O + E * 2 * F * PD

A0T_O = 0
A1T_O = 256
S0_O = 512
S1_O = 528
ABUF = 544


_DNUMS = lax.GatherDimensionNumbers(
    offset_dims=(), collapsed_slice_dims=(0,), start_index_map=(0,))


def _take(v, idx):
    return lax.gather(v, idx[:, None], _DNUMS, (1,),
                      mode=lax.GatherScatterMode.PROMISE_IN_BOUNDS)


def _sc_body(x_hbm, noise_hbm, wts_hbm, out_hbm, disp_hbm, parts_hbm,
             xs, ns, ws, vb, ab, acc, ob, db, sem):
    c = lax.axis_index("c")
    s = lax.axis_index("s")
    wid = s * NC + c
    base = wid * TW

    cp = pltpu.async_copy(x_hbm.at[pl.ds(base * L, TW * L)], xs, sem)
    pltpu.sync_copy(wts_hbm, ws)
    pltpu.sync_copy(noise_hbm.at[pl.ds(base, TW)], ns)

    iota = lax.iota(jnp.int32, 16)
    zero = jnp.zeros((16,), jnp.float32)
    t17 = iota * 17

    for half in range(2):
        for e in range(E):
            a = zero
            for ch in range(F):
                a = a + ws[pl.ds(EW_O + e * 1024 + (half * F + ch) * 16, 16)]
            vb[pl.ds(e * 17, 16)] = a
        for e in range(E, 16):
            vb[pl.ds(e * 17, 16)] = zero
        for k in range(PD):
            ab[pl.ds(half * 256 + k * 16, 16)] = plsc.load_gather(vb, [t17 + k])

    s0 = zero
    s1 = zero
    for e in range(E):
        r0 = ws[pl.ds(EB_O + e * 64, 16)] + ws[pl.ds(EB_O + e * 64 + 16, 16)]
        r1 = ws[pl.ds(EB_O + e * 64 + 32, 16)] + ws[pl.ds(EB_O + e * 64 + 48, 16)]
        s0 = jnp.where(iota == e, jnp.sum(r0), s0)
        s1 = jnp.where(iota == e, jnp.sum(r1), s1)
    ab[pl.ds(S0_O, 16)] = s0 * float(P)
    ab[pl.ds(S1_O, 16)] = s1 * float(P)

    for i in range(16):
        acc[pl.ds(i * 16, 16)] = zero

    cp.wait()

    def group(g, _):
        gi = g * 16
        for t in range(16):
            a = xs[pl.ds(g * 12288 + t * L, 16)]
            for p in range(1, P):
                a = a + xs[pl.ds(g * 12288 + t * L + p * 16, 16)]
            vb[pl.ds(t * 17, 16)] = a
        vt = [plsc.load_gather(vb, [t17 + k]) for k in range(PD)]

        nz = ns[pl.ds(gi, 16)]
        gbrow = ws[pl.ds(GB_O, 16)]
        hs = []
        for e in range(E):
            gwrow = ws[pl.ds(GW_O + e * 16, 16)]
            he = _take(gbrow, jnp.full((16,), e, jnp.int32)) * float(P) + nz
            for k in range(PD):
                he = he + _take(gwrow, jnp.full((16,), k, jnp.int32)) * vt[k]
            hs.append(he)

        m = hs[0]
        ei = jnp.zeros((16,), jnp.int32)
        for e in range(1, E):
            gt = hs[e] > m
            m = jnp.where(gt, hs[e], m)
            ei = jnp.where(gt, e, ei)

        for e in range(E):
            de = jnp.where(ei == e, 1.0, 0.0).astype(jnp.float32)
            plsc.store_scatter(db, [iota * 8 + (g * 128 + e)], de)
            acc[pl.ds(e * 16, 16)] = acc[pl.ds(e * 16, 16)] + hs[e]
            acc[pl.ds(128 + e * 16, 16)] = acc[pl.ds(128 + e * 16, 16)] + de

        t0 = _take(ab[pl.ds(S0_O, 16)], ei)
        t1 = _take(ab[pl.ds(S1_O, 16)], ei)
        for k in range(PD):
            t0 = t0 + vt[k] * _take(ab[pl.ds(A0T_O + k * 16, 16)], ei)
            t1 = t1 + vt[k] * _take(ab[pl.ds(A1T_O + k * 16, 16)], ei)
        plsc.store_scatter(ob, [iota * 2 + (g * 32)], m * t0)
        plsc.store_scatter(ob, [iota * 2 + (g * 32 + 1)], m * t1)
        return 0

    lax.fori_loop(0, NG, group, 0)

    pltpu.sync_copy(ob, out_hbm.at[pl.ds(wid * 256, 256)])
    pltpu.sync_copy(db, disp_hbm.at[pl.ds(wid * 1024, 1024)])
    pltpu.sync_copy(acc, parts_hbm.at[pl.ds(wid * 256, 256)])


_sc_call = pl.kernel(
    _sc_body,
    out_type=[
        jax.ShapeDtypeStruct((B * 2,), jnp.float32),
        jax.ShapeDtypeStruct((B * E,), jnp.float32),
        jax.ShapeDtypeStruct((NW * 256,), jnp.float32),
    ],
    mesh=plsc.VectorSubcoreMesh(core_axis_name="c", subcore_axis_name="s"),
    compiler_params=pltpu.CompilerParams(needs_layout_passes=False),
    scratch_types=[
        pltpu.VMEM((TW * L,), jnp.float32),
        pltpu.VMEM((TW,), jnp.float32),
        pltpu.VMEM((WTS,), jnp.float32),
        pltpu.VMEM((272,), jnp.float32),
        pltpu.VMEM((ABUF,), jnp.float32),
        pltpu.VMEM((256,), jnp.float32),
        pltpu.VMEM((256,), jnp.float32),
        pltpu.VMEM((1024,), jnp.float32),
        pltpu.SemaphoreType.DMA,
    ],
)


def _fin_body(parts_ref, loss_ref):
    a = parts_ref[...]
    ssum = jnp.sum(a, axis=0)[None, :]
    r = lax.broadcasted_iota(jnp.int32, (256, 16), 0)
    cc = lax.broadcasted_iota(jnp.int32, (256, 16), 1)
    sel = (((r // 16) % 8 == cc % 8) & ((r < 128) == (cc < 8))).astype(jnp.float32)
    s2 = jnp.dot(ssum, sel, preferred_element_type=jnp.float32)
    i2 = lax.broadcasted_iota(jnp.int32, (16, 16), 0)
    j2 = lax.broadcasted_iota(jnp.int32, (16, 16), 1)
    sh = (i2 == j2 + 8).astype(jnp.float32)
    s3 = jnp.dot(s2, sh, preferred_element_type=jnp.float32)
    loss_ref[...] = (float(E) / float(B * B)) * jnp.sum(s2 * s3).reshape(1, 1)


def _finalize(parts2):
    return pl.pallas_call(
        _fin_body,
        out_shape=jax.ShapeDtypeStruct((1, 1), jnp.float32),
    )(parts2)


@jax.jit
def _run(xf, noise, wts):
    out_f, disp_f, parts = _sc_call(xf, noise, wts)
    loss = _finalize(parts.reshape(NW, 256))
    return out_f.reshape(B, 2), disp_f.reshape(B, E), loss[0, 0]


def kernel(x, gw, gb, ew, eb):
    b = x.shape[0]
    xf = x.reshape(b * L)
    noise = jax.random.uniform(jax.random.key(42), (b, 1), dtype=jnp.float32).reshape(b)
    gb16 = jnp.pad(gb, (0, 8))
    eb16 = jnp.pad(eb.reshape(E * 2 * F), (0, 512))
    wts = jnp.concatenate([
        gw[:, 0, :].reshape(E * PD),
        gb16,
        eb16,
        ew[:, :, 0, :].reshape(E * 2 * F * PD),
    ])
    return _run(xf, noise, wts)


# speedup vs baseline: 1.9725x; 1.2994x over previous
import functools
import numpy as np
import jax
import jax.numpy as jnp
from jax import lax
from jax.experimental import pallas as pl
from jax.experimental.pallas import tpu as pltpu
from jax.experimental.pallas import tpu_sc as plsc

B = 4096
L = 768
E = 8
P = 48
PD = 16
F = 32

NC = 2
NS = 16
NW = NC * NS
TW = B // NW
NG = TW // 16
XW = TW * L
NCH = 4
CH = XW // NCH

GW_O = 0
GB_O = 128
EB_O = 136
EW_O = 136 + 512
WTS = EW_O + E * 2 * F * PD

A0T_O = 0
A1T_O = 256
S0_O = 512
S1_O = 528
ABUF = 544

_DNUMS = lax.GatherDimensionNumbers(
    offset_dims=(), collapsed_slice_dims=(0,), start_index_map=(0,))


def _take(v, idx):
    return lax.gather(v, idx[:, None], _DNUMS, (1,),
                      mode=lax.GatherScatterMode.PROMISE_IN_BOUNDS)


def _treesum(vs):
    while len(vs) > 1:
        nxt = [vs[i] + vs[i + 1] for i in range(0, len(vs) - 1, 2)]
        if len(vs) % 2:
            nxt.append(vs[-1])
        vs = nxt
    return vs[0]


def _sc_body(x_hbm, noise_hbm, wts_hbm, out_hbm, disp_hbm, parts_hbm,
             xs, ns, ws, vb, ab, acc, ob, db,
             sem0, sem1, sem2, sem3):
    c = lax.axis_index("c")
    s = lax.axis_index("s")
    wid = s * NC + c
    base = wid * TW

    sems = [sem0, sem1, sem2, sem3]
    cps = [
        pltpu.async_copy(x_hbm.at[pl.ds(base * L + i * CH, CH)],
                         xs.at[pl.ds(i * CH, CH)], sems[i])
        for i in range(NCH)
    ]
    pltpu.sync_copy(wts_hbm, ws)
    pltpu.sync_copy(noise_hbm.at[pl.ds(base, TW)], ns)

    iota = lax.iota(jnp.int32, 16)
    zero = jnp.zeros((16,), jnp.float32)
    t17 = iota * 17

    for e in range(E, 16):
        vb[pl.ds(e * 17, 16)] = zero
    for half in range(2):
        def prep_e(e, _):
            a = _treesum([ws[pl.ds(EW_O + e * 1024 + (half * F + ch) * 16, 16)]
                          for ch in range(F)])
            vb[pl.ds(e * 17, 16)] = a
            return 0
        lax.fori_loop(0, E, prep_e, 0)
        for k in range(PD):
            ab[pl.ds(half * 256 + k * 16, 16)] = plsc.load_gather(vb, [t17 + k])

    def prep_s(e, carry):
        s0, s1 = carry
        r0 = ws[pl.ds(EB_O + e * 64, 16)] + ws[pl.ds(EB_O + e * 64 + 16, 16)]
        r1 = ws[pl.ds(EB_O + e * 64 + 32, 16)] + ws[pl.ds(EB_O + e * 64 + 48, 16)]
        s0 = jnp.where(iota == e, jnp.sum(r0), s0)
        s1 = jnp.where(iota == e, jnp.sum(r1), s1)
        return (s0, s1)

    s0, s1 = lax.fori_loop(0, E, prep_s, (zero, zero))
    ab[pl.ds(S0_O, 16)] = s0 * float(P)
    ab[pl.ds(S1_O, 16)] = s1 * float(P)

    for i in range(16):
        acc[pl.ds(i * 16, 16)] = zero

    def group(g, _):
        gi = g * 16

        def tok(t, _):
            o = g * 12288 + t * L
            vb[pl.ds(t * 17, 16)] = _treesum(
                [xs[pl.ds(o + p * 16, 16)] for p in range(P)])
            return 0

        lax.fori_loop(0, 16, tok, 0)

        vt = [plsc.load_gather(vb, [t17 + k]) for k in range(PD)]

        nz = ns[pl.ds(gi, 16)]
        gbrow = ws[pl.ds(GB_O, 16)]

        def e_loop(e, carry):
            m, ei = carry
            gwrow = ws[pl.ds(GW_O + e * 16, 16)]
            he = _take(gbrow, jnp.broadcast_to(e, (16,))) * float(P) + nz
            for k in range(PD):
                he = he + _take(gwrow, jnp.full((16,), k, jnp.int32)) * vt[k]
            acc[pl.ds(e * 16, 16)] = acc[pl.ds(e * 16, 16)] + he
            gt = he > m
            return (jnp.where(gt, he, m), jnp.where(gt, e, ei))

        m, ei = lax.fori_loop(
            0, E, e_loop,
            (jnp.full((16,), -jnp.inf, jnp.float32), jnp.zeros((16,), jnp.int32)))

        def d_loop(e, _):
            de = jnp.where(ei == e, 1.0, 0.0).astype(jnp.float32)
            plsc.store_scatter(db, [(gi + iota) * E + e], de)
            acc[pl.ds(128 + e * 16, 16)] = acc[pl.ds(128 + e * 16, 16)] + de
            return 0

        lax.fori_loop(0, E, d_loop, 0)

        t0 = _take(ab[pl.ds(S0_O, 16)], ei)
        t1 = _take(ab[pl.ds(S1_O, 16)], ei)
        for k in range(PD):
            t0 = t0 + vt[k] * _take(ab[pl.ds(A0T_O + k * 16, 16)], ei)
            t1 = t1 + vt[k] * _take(ab[pl.ds(A1T_O + k * 16, 16)], ei)
        plsc.store_scatter(ob, [(gi + iota) * 2], m * t0)
        plsc.store_scatter(ob, [(gi + iota) * 2 + 1], m * t1)
        return 0

    for ch in range(NCH):
        cps[ch].wait()
        lax.fori_loop(ch * (NG // NCH), (ch + 1) * (NG // NCH), group, 0)

    pltpu.sync_copy(ob, out_hbm.at[pl.ds(base * 2, TW * 2)])
    pltpu.sync_copy(db, disp_hbm.at[pl.ds(base * E, TW * E)])
    pltpu.sync_copy(acc, parts_hbm.at[pl.ds(wid * 256, 256)])


_sc_call = pl.kernel(
    _sc_body,
    out_type=[
        jax.ShapeDtypeStruct((B * 2,), jnp.float32),
        jax.ShapeDtypeStruct((B * E,), jnp.float32),
        jax.ShapeDtypeStruct((NW * 256,), jnp.float32),
    ],
    mesh=plsc.VectorSubcoreMesh(core_axis_name="c", subcore_axis_name="s"),
    compiler_params=pltpu.CompilerParams(needs_layout_passes=False),
    scratch_types=[
        pltpu.VMEM((XW,), jnp.float32),
        pltpu.VMEM((TW,), jnp.float32),
        pltpu.VMEM((WTS,), jnp.float32),
        pltpu.VMEM((272,), jnp.float32),
        pltpu.VMEM((ABUF,), jnp.float32),
        pltpu.VMEM((256,), jnp.float32),
        pltpu.VMEM((TW * 2,), jnp.float32),
        pltpu.VMEM((TW * E,), jnp.float32),
        pltpu.SemaphoreType.DMA,
        pltpu.SemaphoreType.DMA,
        pltpu.SemaphoreType.DMA,
        pltpu.SemaphoreType.DMA,
    ],
)


def _fin_body(parts_ref, loss_ref):
    a = parts_ref[...]
    ssum = jnp.sum(a, axis=0)[None, :]
    r = lax.broadcasted_iota(jnp.int32, (256, 16), 0)
    cc = lax.broadcasted_iota(jnp.int32, (256, 16), 1)
    sel = (((r // 16) % 8 == cc % 8) & ((r < 128) == (cc < 8))).astype(jnp.float32)
    s2 = jnp.dot(ssum, sel, preferred_element_type=jnp.float32)
    i2 = lax.broadcasted_iota(jnp.int32, (16, 16), 0)
    j2 = lax.broadcasted_iota(jnp.int32, (16, 16), 1)
    sh = (i2 == j2 + 8).astype(jnp.float32)
    s3 = jnp.dot(s2, sh, preferred_element_type=jnp.float32)
    loss_ref[...] = (float(E) / float(B * B)) * jnp.sum(s2 * s3).reshape(1, 1)


@jax.jit
def _run(xf, noise, wts):
    out_f, disp_f, parts = _sc_call(xf, noise, wts)
    loss = pl.pallas_call(
        _fin_body,
        out_shape=jax.ShapeDtypeStruct((1, 1), jnp.float32),
    )(parts.reshape(NW, 256))
    return out_f.reshape(B, 2), disp_f.reshape(B, E), loss[0, 0]


def kernel(x, gw, gb, ew, eb):
    b = x.shape[0]
    xf = x.reshape(b * L)
    wts = jnp.concatenate([
        gw[:, 0, :].reshape(E * PD),
        gb,
        eb.reshape(E * 2 * F),
        ew[:, :, 0, :].reshape(E * 2 * F * PD),
    ])
    noise = jax.random.uniform(jax.random.key(42), (b, 1), dtype=jnp.float32).reshape(b)
    return _run(xf, noise, wts)


# speedup vs baseline: 2.3124x; 1.1723x over previous
import functools
import numpy as np
import jax
import jax.numpy as jnp
from jax import lax
from jax.experimental import pallas as pl
from jax.experimental.pallas import tpu as pltpu
from jax.experimental.pallas import tpu_sc as plsc

B = 4096
L = 768
E = 8
P = 48
PD = 16
F = 32

NC = 2
NS = 16
NW = NC * NS
TW = B // NW
NG = TW // 16
XW = TW * L
NCH = 4
CH = XW // NCH

GW_O = 0
GB_O = 128
EB_O = 136
EW_O = 136 + 512
WTS = EW_O + E * 2 * F * PD

A0T_O = 0
A1T_O = 256
S0_O = 512
S1_O = 528
ABUF = 544

_DNUMS = lax.GatherDimensionNumbers(
    offset_dims=(), collapsed_slice_dims=(0,), start_index_map=(0,))


def _take(v, idx):
    return lax.gather(v, idx[:, None], _DNUMS, (1,),
                      mode=lax.GatherScatterMode.PROMISE_IN_BOUNDS)


def _treesum(vs):
    while len(vs) > 1:
        nxt = [vs[i] + vs[i + 1] for i in range(0, len(vs) - 1, 2)]
        if len(vs) % 2:
            nxt.append(vs[-1])
        vs = nxt
    return vs[0]


def _sc_body(x_hbm, noise_hbm, wts_hbm, out_hbm, disp_hbm, parts_hbm,
             xs, ns, ws, vb, ab, acc, ob, db,
             sem0, sem1, sem2, sem3):
    c = lax.axis_index("c")
    s = lax.axis_index("s")
    wid = s * NC + c
    base = wid * TW

    sems = [sem0, sem1, sem2, sem3]
    cps = [
        pltpu.async_copy(x_hbm.at[pl.ds(base * L + i * CH, CH)],
                         xs.at[pl.ds(i * CH, CH)], sems[i])
        for i in range(NCH)
    ]
    pltpu.sync_copy(wts_hbm, ws)
    pltpu.sync_copy(noise_hbm.at[pl.ds(base, TW)], ns)

    iota = lax.iota(jnp.int32, 16)
    zero = jnp.zeros((16,), jnp.float32)
    t17 = iota * 17

    for e in range(E, 16):
        vb[pl.ds(e * 17, 16)] = zero
    for half in range(2):
        def prep_e(e, _):
            a = _treesum([ws[pl.ds(EW_O + e * 1024 + (half * F + ch) * 16, 16)]
                          for ch in range(F)])
            vb[pl.ds(e * 17, 16)] = a
            return 0
        lax.fori_loop(0, E, prep_e, 0)
        for k in range(PD):
            ab[pl.ds(half * 256 + k * 16, 16)] = plsc.load_gather(vb, [t17 + k])

    def prep_s(e, carry):
        s0, s1 = carry
        r0 = ws[pl.ds(EB_O + e * 64, 16)] + ws[pl.ds(EB_O + e * 64 + 16, 16)]
        r1 = ws[pl.ds(EB_O + e * 64 + 32, 16)] + ws[pl.ds(EB_O + e * 64 + 48, 16)]
        s0 = jnp.where(iota == e, jnp.sum(r0), s0)
        s1 = jnp.where(iota == e, jnp.sum(r1), s1)
        return (s0, s1)

    s0, s1 = lax.fori_loop(0, E, prep_s, (zero, zero))
    ab[pl.ds(S0_O, 16)] = s0 * float(P)
    ab[pl.ds(S1_O, 16)] = s1 * float(P)

    for i in range(16):
        acc[pl.ds(i * 16, 16)] = zero

    def group(g, _):
        gi = g * 16

        def tok(i, _):
            for u in range(2):
                t = i * 2 + u
                o = g * 12288 + t * L
                vb[pl.ds(t * 17, 16)] = _treesum(
                    [xs[pl.ds(o + p * 16, 16)] for p in range(P)])
            return 0

        lax.fori_loop(0, 8, tok, 0)

        vt = [plsc.load_gather(vb, [t17 + k]) for k in range(PD)]

        nz = ns[pl.ds(gi, 16)]
        gbrow = ws[pl.ds(GB_O, 16)]

        def e_loop(e, carry):
            m, ei = carry
            gwrow = ws[pl.ds(GW_O + e * 16, 16)]
            he = _take(gbrow, jnp.broadcast_to(e, (16,))) * float(P) + nz
            for k in range(PD):
                he = he + _take(gwrow, jnp.full((16,), k, jnp.int32)) * vt[k]
            acc[pl.ds(e * 16, 16)] = acc[pl.ds(e * 16, 16)] + he
            gt = he > m
            return (jnp.where(gt, he, m), jnp.where(gt, e, ei))

        m, ei = lax.fori_loop(
            0, E, e_loop,
            (jnp.full((16,), -jnp.inf, jnp.float32), jnp.zeros((16,), jnp.int32)))

        def d_loop(e, _):
            de = jnp.where(ei == e, 1.0, 0.0).astype(jnp.float32)
            plsc.store_scatter(db, [jnp.broadcast_to(e, (16,)), gi + iota], de)
            acc[pl.ds(128 + e * 16, 16)] = acc[pl.ds(128 + e * 16, 16)] + de
            return 0

        lax.fori_loop(0, E, d_loop, 0)

        t0 = _take(ab[pl.ds(S0_O, 16)], ei)
        t1 = _take(ab[pl.ds(S1_O, 16)], ei)
        for k in range(PD):
            t0 = t0 + vt[k] * _take(ab[pl.ds(A0T_O + k * 16, 16)], ei)
            t1 = t1 + vt[k] * _take(ab[pl.ds(A1T_O + k * 16, 16)], ei)
        zi = jnp.zeros((16,), jnp.int32)
        plsc.store_scatter(ob, [zi, gi + iota], m * t0)
        plsc.store_scatter(ob, [zi + 1, gi + iota], m * t1)
        return 0

    for ch in range(NCH):
        cps[ch].wait()
        lax.fori_loop(ch * (NG // NCH), (ch + 1) * (NG // NCH), group, 0)

    for j in range(2):
        pltpu.sync_copy(ob.at[pl.ds(j, 1), :],
                        out_hbm.at[pl.ds(j, 1), pl.ds(base, TW)])
    for e in range(E):
        pltpu.sync_copy(db.at[pl.ds(e, 1), :],
                        disp_hbm.at[pl.ds(e, 1), pl.ds(base, TW)])
    pltpu.sync_copy(acc, parts_hbm.at[wid])


_sc_call = pl.kernel(
    _sc_body,
    out_type=[
        jax.ShapeDtypeStruct((2, B), jnp.float32),
        jax.ShapeDtypeStruct((E, B), jnp.float32),
        jax.ShapeDtypeStruct((NW, 256), jnp.float32),
    ],
    mesh=plsc.VectorSubcoreMesh(core_axis_name="c", subcore_axis_name="s"),
    compiler_params=pltpu.CompilerParams(needs_layout_passes=False),
    scratch_types=[
        pltpu.VMEM((XW,), jnp.float32),
        pltpu.VMEM((TW,), jnp.float32),
        pltpu.VMEM((WTS,), jnp.float32),
        pltpu.VMEM((272,), jnp.float32),
        pltpu.VMEM((ABUF,), jnp.float32),
        pltpu.VMEM((256,), jnp.float32),
        pltpu.VMEM((2, TW), jnp.float32),
        pltpu.VMEM((E, TW), jnp.float32),
        pltpu.SemaphoreType.DMA,
        pltpu.SemaphoreType.DMA,
        pltpu.SemaphoreType.DMA,
        pltpu.SemaphoreType.DMA,
    ],
)


def _fin_body(parts_ref, loss_ref):
    a = parts_ref[...]
    ssum = jnp.sum(a, axis=0)[None, :]
    r = lax.broadcasted_iota(jnp.int32, (256, 16), 0)
    cc = lax.broadcasted_iota(jnp.int32, (256, 16), 1)
    sel = (((r // 16) % 8 == cc % 8) & ((r < 128) == (cc < 8))).astype(jnp.float32)
    s2 = jnp.dot(ssum, sel, preferred_element_type=jnp.float32)
    i2 = lax.broadcasted_iota(jnp.int32, (16, 16), 0)
    j2 = lax.broadcasted_iota(jnp.int32, (16, 16), 1)
    sh = (i2 == j2 + 8).astype(jnp.float32)
    s3 = jnp.dot(s2, sh, preferred_element_type=jnp.float32)
    loss_ref[...] = (float(E) / float(B * B)) * jnp.sum(s2 * s3).reshape(1, 1)


@jax.jit
def _run(xf, noise, wts):
    out_t, disp_t, parts = _sc_call(xf, noise, wts)
    loss = pl.pallas_call(
        _fin_body,
        out_shape=jax.ShapeDtypeStruct((1, 1), jnp.float32),
    )(parts)
    return out_t.T, disp_t.T, loss[0, 0]


def kernel(x, gw, gb, ew, eb):
    b = x.shape[0]
    xf = x.reshape(b * L)
    wts = jnp.concatenate([
        gw[:, 0, :].reshape(E * PD),
        gb,
        eb.reshape(E * 2 * F),
        ew[:, :, 0, :].reshape(E * 2 * F * PD),
    ])
    noise = jax.random.uniform(jax.random.key(42), (b, 1), dtype=jnp.float32).reshape(b)
    return _run(xf, noise, wts)
